# baseline (speedup 1.0000x reference)
import functools

import jax
import jax.numpy as jnp
from jax import lax
from jax.experimental import pallas as pl
from jax.experimental.pallas import tpu as pltpu
from jax.experimental.pallas import tpu_sc as plsc

_D = 64
_NC, _NS = 2, 16
_NW = _NC * _NS


@functools.cache
def _gather_call(n_rows: int, chunk: int):
    n_per_w = n_rows // _NW
    n_chunks = n_per_w // chunk
    mesh = plsc.VectorSubcoreMesh(core_axis_name="c", subcore_axis_name="s")

    @functools.partial(
        pl.kernel,
        out_type=jax.ShapeDtypeStruct((n_rows, _D), jnp.float32),
        mesh=mesh,
        scratch_types=[
            pltpu.VMEM((chunk,), jnp.int32),
            pltpu.VMEM((chunk, _D), jnp.float32),
            pltpu.SemaphoreType.DMA,
        ],
        compiler_params=pltpu.CompilerParams(use_tc_tiling_on_sc=False),
    )
    def k(idx_hbm, table_hbm, out_hbm, idx_v, rows_v, sem):
        wid = lax.axis_index("s") * _NC + lax.axis_index("c")
        base = wid * n_per_w

        @pl.loop(0, n_chunks)
        def _(i):
            off = base + i * chunk
            pltpu.sync_copy(idx_hbm.at[pl.ds(off, chunk)], idx_v)
            pltpu.async_copy(table_hbm.at[idx_v], rows_v, sem).wait()
            pltpu.sync_copy(rows_v, out_hbm.at[pl.ds(off, chunk)])

    return k


def kernel(token_ids, embedding_matrix):
    b, s = token_ids.shape
    n = b * s
    idx = token_ids.reshape(n).astype(jnp.int32)
    out = _gather_call(n, 1024)(idx, embedding_matrix)
    return out.reshape(b, s, _D)


# speedup vs baseline: 1.0144x; 1.0144x over previous
import functools

import jax
import jax.numpy as jnp
from jax import lax
from jax.experimental import pallas as pl
from jax.experimental.pallas import tpu as pltpu
from jax.experimental.pallas import tpu_sc as plsc

_D = 64
_NC, _NS = 2, 16
_NW = _NC * _NS
_NBUF = 2


@functools.cache
def _gather_call(n_rows: int, chunk: int):
    n_per_w = n_rows // _NW
    n_chunks = n_per_w // chunk
    assert n_per_w % chunk == 0 and n_chunks % _NBUF == 0
    mesh = plsc.VectorSubcoreMesh(core_axis_name="c", subcore_axis_name="s")

    @functools.partial(
        pl.kernel,
        out_type=jax.ShapeDtypeStruct((n_rows, _D), jnp.float32),
        mesh=mesh,
        scratch_types=[
            pltpu.VMEM((n_per_w,), jnp.int32),
            pltpu.VMEM((chunk, _D), jnp.float32),
            pltpu.VMEM((chunk, _D), jnp.float32),
            pltpu.SemaphoreType.DMA,
            pltpu.SemaphoreType.DMA,
            pltpu.SemaphoreType.DMA,
            pltpu.SemaphoreType.DMA,
        ],
        compiler_params=pltpu.CompilerParams(use_tc_tiling_on_sc=False),
    )
    def k(idx_hbm, table_hbm, out_hbm, idx_v, r0, r1, g0, g1, s0, s1):
        wid = lax.axis_index("s") * _NC + lax.axis_index("c")
        base = wid * n_per_w
        rows = (r0, r1)
        gsem = (g0, g1)
        ssem = (s0, s1)

        pltpu.sync_copy(idx_hbm.at[pl.ds(base, n_per_w)], idx_v)

        for b in range(_NBUF):
            pltpu.async_copy(
                table_hbm.at[idx_v.at[pl.ds(b * chunk, chunk)]], rows[b], gsem[b])

        n_steps = n_chunks // _NBUF

        @pl.loop(0, n_steps)
        def _(p):
            i0 = p * _NBUF
            for b in range(_NBUF):
                i = i0 + b
                pltpu.make_async_copy(
                    table_hbm.at[idx_v.at[pl.ds(0, chunk)]], rows[b], gsem[b]).wait()
                pltpu.async_copy(
                    rows[b], out_hbm.at[pl.ds(base + i * chunk, chunk)], ssem[b])

                @pl.when(p < n_steps - 1)
                def _():
                    pltpu.make_async_copy(
                        rows[b], out_hbm.at[pl.ds(base, chunk)], ssem[b]).wait()
                    pltpu.async_copy(
                        table_hbm.at[idx_v.at[pl.ds((i + _NBUF) * chunk, chunk)]],
                        rows[b], gsem[b])

        for b in range(_NBUF):
            pltpu.make_async_copy(
                rows[b], out_hbm.at[pl.ds(base, chunk)], ssem[b]).wait()

    return k


def kernel(token_ids, embedding_matrix):
    b, s = token_ids.shape
    n = b * s
    idx = token_ids.reshape(n).astype(jnp.int32)
    out = _gather_call(n, 800)(idx, embedding_matrix)
    return out.reshape(b, s, _D)


# speedup vs baseline: 1.0687x; 1.0536x over previous
import functools

import jax
import jax.numpy as jnp
from jax import lax
from jax.experimental import pallas as pl
from jax.experimental.pallas import tpu as pltpu
from jax.experimental.pallas import tpu_sc as plsc

_D = 64
_NC, _NS = 2, 16
_NW = _NC * _NS
_NBUF = 2


@functools.cache
def _gather_call(n_rows: int, chunk: int):
    n_per_w = n_rows // _NW
    n_chunks = n_per_w // chunk
    assert n_per_w % chunk == 0 and n_chunks % _NBUF == 0
    mesh = plsc.VectorSubcoreMesh(core_axis_name="c", subcore_axis_name="s")

    @functools.partial(
        pl.kernel,
        out_type=jax.ShapeDtypeStruct((n_rows, _D), jnp.float32),
        mesh=mesh,
        scratch_types=[
            pltpu.VMEM((n_per_w,), jnp.int32),
            pltpu.VMEM((chunk, _D), jnp.float32),
            pltpu.VMEM((chunk, _D), jnp.float32),
            pltpu.SemaphoreType.DMA,
            pltpu.SemaphoreType.DMA,
            pltpu.SemaphoreType.DMA,
            pltpu.SemaphoreType.DMA,
        ],
        compiler_params=pltpu.CompilerParams(use_tc_tiling_on_sc=False),
    )
    def k(idx_hbm, table_hbm, out_hbm, idx_v, r0, r1, g0, g1, s0, s1):
        wid = lax.axis_index("s") * _NC + lax.axis_index("c")
        base = wid * n_per_w
        rows = (r0, r1)
        gsem = (g0, g1)
        ssem = (s0, s1)

        pltpu.sync_copy(idx_hbm.at[pl.ds(base, n_per_w)], idx_v)

        for b in range(_NBUF):
            pltpu.async_copy(
                table_hbm.at[idx_v.at[pl.ds(b * chunk, chunk)]], rows[b], gsem[b])

        n_steps = n_chunks // _NBUF

        @pl.loop(0, n_steps)
        def _(p):
            i0 = p * _NBUF
            for b in range(_NBUF):
                i = i0 + b
                pltpu.make_async_copy(
                    table_hbm.at[idx_v.at[pl.ds(0, chunk)]], rows[b], gsem[b]).wait()
                pltpu.async_copy(
                    rows[b], out_hbm.at[pl.ds(base + i * chunk, chunk)], ssem[b])

                @pl.when(p < n_steps - 1)
                def _():
                    pltpu.make_async_copy(
                        rows[b], out_hbm.at[pl.ds(base, chunk)], ssem[b]).wait()
                    pltpu.async_copy(
                        table_hbm.at[idx_v.at[pl.ds((i + _NBUF) * chunk, chunk)]],
                        rows[b], gsem[b])

        for b in range(_NBUF):
            pltpu.make_async_copy(
                rows[b], out_hbm.at[pl.ds(base, chunk)], ssem[b]).wait()

    return k


def kernel(token_ids, embedding_matrix):
    b, s = token_ids.shape
    n = b * s
    idx = token_ids.reshape(n).astype(jnp.int32) * 2
    table_padded = jnp.pad(embedding_matrix, ((0, 0), (0, _D)))
    table2 = table_padded.reshape(2 * embedding_matrix.shape[0], _D)
    out = _gather_call(n, 800)(idx, table2)
    return out.reshape(b, s, _D)


# speedup vs baseline: 1.4506x; 1.3573x over previous
import functools

import jax
import jax.numpy as jnp
from jax import lax
from jax.experimental import pallas as pl
from jax.experimental.pallas import tpu as pltpu
from jax.experimental.pallas import tpu_sc as plsc

_D = 64
_NC, _NS = 2, 16
_NW = _NC * _NS
_NBUF = 2


@functools.cache
def _gather_call(n_rows: int, chunk: int):
    n_per_w = n_rows // _NW
    n_chunks = n_per_w // chunk
    assert n_per_w % chunk == 0 and n_chunks % _NBUF == 0
    mesh = plsc.VectorSubcoreMesh(core_axis_name="c", subcore_axis_name="s")

    @functools.partial(
        pl.kernel,
        out_type=jax.ShapeDtypeStruct((n_rows, 2 * _D), jnp.float32),
        mesh=mesh,
        scratch_types=[
            pltpu.VMEM((n_per_w,), jnp.int32),
            pltpu.VMEM((chunk, _D), jnp.float32),
            pltpu.VMEM((chunk, _D), jnp.float32),
            pltpu.SemaphoreType.DMA,
            pltpu.SemaphoreType.DMA,
            pltpu.SemaphoreType.DMA,
            pltpu.SemaphoreType.DMA,
        ],
        compiler_params=pltpu.CompilerParams(use_tc_tiling_on_sc=False),
    )
    def k(idx_hbm, table_hbm, out_hbm, idx_v, r0, r1, g0, g1, s0, s1):
        wid = lax.axis_index("s") * _NC + lax.axis_index("c")
        base = wid * n_per_w
        rows = (r0, r1)
        gsem = (g0, g1)
        ssem = (s0, s1)

        pltpu.sync_copy(idx_hbm.at[pl.ds(base, n_per_w)], idx_v)

        for b in range(_NBUF):
            pltpu.async_copy(
                table_hbm.at[idx_v.at[pl.ds(b * chunk, chunk)]], rows[b], gsem[b])

        n_steps = n_chunks // _NBUF

        @pl.loop(0, n_steps)
        def _(p):
            i0 = p * _NBUF
            for b in range(_NBUF):
                i = i0 + b
                pltpu.make_async_copy(
                    table_hbm.at[idx_v.at[pl.ds(0, chunk)]], rows[b], gsem[b]).wait()
                pltpu.async_copy(
                    rows[b],
                    out_hbm.at[pl.ds(base + i * chunk, chunk), pl.ds(0, _D)],
                    ssem[b])

                @pl.when(p < n_steps - 1)
                def _():
                    pltpu.make_async_copy(
                        rows[b], out_hbm.at[pl.ds(base, chunk), pl.ds(0, _D)],
                        ssem[b]).wait()
                    pltpu.async_copy(
                        table_hbm.at[idx_v.at[pl.ds((i + _NBUF) * chunk, chunk)]],
                        rows[b], gsem[b])

        for b in range(_NBUF):
            pltpu.make_async_copy(
                rows[b], out_hbm.at[pl.ds(base, chunk), pl.ds(0, _D)],
                ssem[b]).wait()

    return k


def kernel(token_ids, embedding_matrix):
    b, s = token_ids.shape
    n = b * s
    idx = token_ids.reshape(n).astype(jnp.int32) * 2
    table_padded = jnp.pad(embedding_matrix, ((0, 0), (0, _D)))
    table2 = table_padded.reshape(2 * embedding_matrix.shape[0], _D)
    out_pad = _gather_call(n, 800)(idx, table2)
    return out_pad[:, :_D].reshape(b, s, _D)
